# speedup vs baseline: 41.6169x; 41.6169x over previous
import jax
import jax.numpy as jnp
from jax.experimental import pallas as pl

_LAM = 0.5
_TGT = 1.0 - _LAM
_N_BISECT = 16
_N_NEWTON = 4


def _sparsegen_rows_kernel(x_ref, o_ref):
    z = x_ref[:]
    m = jnp.max(z, axis=-1, keepdims=True)
    w = jnp.maximum(z - (m - _TGT), 0.0)

    def bisect_body(_, carry):
        dlo, dhi = carry
        mid = 0.5 * (dlo + dhi)
        f = jnp.sum(jnp.maximum(w - mid, 0.0), axis=-1, keepdims=True)
        gt = f > _TGT
        return jnp.where(gt, mid, dlo), jnp.where(gt, dhi, mid)

    dlo0 = jnp.zeros_like(m)
    dhi0 = jnp.full_like(m, _TGT)
    dlo, _ = jax.lax.fori_loop(0, _N_BISECT, bisect_body, (dlo0, dhi0))

    def newton_body(_, d):
        mask = w > d
        k = jnp.sum(mask.astype(z.dtype), axis=-1, keepdims=True)
        s = jnp.sum(jnp.where(mask, w, 0.0), axis=-1, keepdims=True)
        return (s - _TGT) / jnp.maximum(k, 1.0)

    d = jax.lax.fori_loop(0, _N_NEWTON, newton_body, dlo)

    o_ref[:] = jnp.maximum((w - d) * (1.0 / _TGT), 0.0)


@jax.jit
def kernel(input):
    b, q, n = input.shape
    rows = b * q
    x2 = input.reshape(rows, n)
    block_rows = 16
    out = pl.pallas_call(
        _sparsegen_rows_kernel,
        out_shape=jax.ShapeDtypeStruct((rows, n), input.dtype),
        grid=(rows // block_rows,),
        in_specs=[pl.BlockSpec((block_rows, n), lambda i: (i, 0))],
        out_specs=pl.BlockSpec((block_rows, n), lambda i: (i, 0)),
    )(x2)
    return out.reshape(b, q, n)


# speedup vs baseline: 53.5015x; 1.2856x over previous
import jax
import jax.numpy as jnp
from jax.experimental import pallas as pl

_LAM = 0.5
_TGT = 1.0 - _LAM
_N_BISECT = 12
_N_NEWTON = 3


def _sparsegen_rows_kernel(x_ref, o_ref):
    z = x_ref[:]
    m = jnp.max(z, axis=-1, keepdims=True)
    w = jnp.maximum(z - (m - _TGT), 0.0)

    def bisect_body(_, carry):
        dlo, dhi = carry
        mid = 0.5 * (dlo + dhi)
        f = jnp.sum(jnp.maximum(w - mid, 0.0), axis=-1, keepdims=True)
        gt = f > _TGT
        return jnp.where(gt, mid, dlo), jnp.where(gt, dhi, mid)

    dlo0 = jnp.zeros_like(m)
    dhi0 = jnp.full_like(m, _TGT)
    dlo, _ = jax.lax.fori_loop(0, _N_BISECT, bisect_body, (dlo0, dhi0))

    def newton_body(_, d):
        mask = w > d
        k = jnp.sum(mask.astype(z.dtype), axis=-1, keepdims=True)
        s = jnp.sum(jnp.where(mask, w, 0.0), axis=-1, keepdims=True)
        return (s - _TGT) / jnp.maximum(k, 1.0)

    d = jax.lax.fori_loop(0, _N_NEWTON, newton_body, dlo)

    o_ref[:] = jnp.maximum((w - d) * (1.0 / _TGT), 0.0)


@jax.jit
def kernel(input):
    b, q, n = input.shape
    rows = b * q
    x2 = input.reshape(rows, n)
    block_rows = 16
    out = pl.pallas_call(
        _sparsegen_rows_kernel,
        out_shape=jax.ShapeDtypeStruct((rows, n), input.dtype),
        grid=(rows // block_rows,),
        in_specs=[pl.BlockSpec((block_rows, n), lambda i: (i, 0))],
        out_specs=pl.BlockSpec((block_rows, n), lambda i: (i, 0)),
    )(x2)
    return out.reshape(b, q, n)


# speedup vs baseline: 53.9043x; 1.0075x over previous
import jax
import jax.numpy as jnp
from jax.experimental import pallas as pl

_LAM = 0.5
_TGT = 1.0 - _LAM
_GRP = 16
_N_BISECT = 12
_N_NEWTON_G = 4
_N_NEWTON_F = 3


def _sparsegen_rows_kernel(x_ref, o_ref):
    z = x_ref[:]
    r, n = z.shape
    g = n // _GRP
    zg = jnp.max(z.reshape(r, _GRP, g), axis=1)
    m = jnp.max(zg, axis=-1, keepdims=True)
    wg = jnp.maximum(zg - (m - _TGT), 0.0)

    def bisect_body(_, carry):
        dlo, dhi = carry
        mid = 0.5 * (dlo + dhi)
        f = jnp.sum(jnp.maximum(wg - mid, 0.0), axis=-1, keepdims=True)
        gt = f > _TGT
        return jnp.where(gt, mid, dlo), jnp.where(gt, dhi, mid)

    dlo0 = jnp.zeros_like(m)
    dhi0 = jnp.full_like(m, _TGT)
    dlo, _ = jax.lax.fori_loop(0, _N_BISECT, bisect_body, (dlo0, dhi0))

    def newton_g_body(_, d):
        mask = wg > d
        k = jnp.sum(mask.astype(z.dtype), axis=-1, keepdims=True)
        s = jnp.sum(jnp.where(mask, wg, 0.0), axis=-1, keepdims=True)
        return (s - _TGT) / jnp.maximum(k, 1.0)

    d = jax.lax.fori_loop(0, _N_NEWTON_G, newton_g_body, dlo)
    t0 = (m - _TGT) + d

    def newton_f_body(_, t):
        mask = z > t
        k = jnp.sum(mask.astype(z.dtype), axis=-1, keepdims=True)
        s = jnp.sum(jnp.where(mask, z, 0.0), axis=-1, keepdims=True)
        return (s - _TGT) / jnp.maximum(k, 1.0)

    tau = jax.lax.fori_loop(0, _N_NEWTON_F, newton_f_body, t0)

    o_ref[:] = jnp.maximum((z - tau) * (1.0 / _TGT), 0.0)


@jax.jit
def kernel(input):
    b, q, n = input.shape
    rows = b * q
    x2 = input.reshape(rows, n)
    block_rows = 16
    out = pl.pallas_call(
        _sparsegen_rows_kernel,
        out_shape=jax.ShapeDtypeStruct((rows, n), input.dtype),
        grid=(rows // block_rows,),
        in_specs=[pl.BlockSpec((block_rows, n), lambda i: (i, 0))],
        out_specs=pl.BlockSpec((block_rows, n), lambda i: (i, 0)),
    )(x2)
    return out.reshape(b, q, n)


# speedup vs baseline: 98.8112x; 1.8331x over previous
import jax
import jax.numpy as jnp
from jax.experimental import pallas as pl

_LAM = 0.5
_TGT = 1.0 - _LAM
_GRP = 16
_N_BISECT = 12
_N_NEWTON_G = 4
_N_NEWTON_F = 3


def _sparsegen_rows_kernel(x_ref, o_ref):
    z = x_ref[:]
    r, n = z.shape
    g = n // _GRP
    zg = z[:, :g]
    for j in range(1, _GRP):
        zg = jnp.maximum(zg, z[:, j * g:(j + 1) * g])
    m = jnp.max(zg, axis=-1, keepdims=True)
    wg = jnp.maximum(zg - (m - _TGT), 0.0)

    def bisect_body(_, carry):
        dlo, dhi = carry
        mid = 0.5 * (dlo + dhi)
        f = jnp.sum(jnp.maximum(wg - mid, 0.0), axis=-1, keepdims=True)
        gt = f > _TGT
        return jnp.where(gt, mid, dlo), jnp.where(gt, dhi, mid)

    dlo0 = jnp.zeros_like(m)
    dhi0 = jnp.full_like(m, _TGT)
    dlo, _ = jax.lax.fori_loop(0, _N_BISECT, bisect_body, (dlo0, dhi0))

    def newton_g_body(_, d):
        mask = wg > d
        k = jnp.sum(mask.astype(z.dtype), axis=-1, keepdims=True)
        s = jnp.sum(jnp.where(mask, wg, 0.0), axis=-1, keepdims=True)
        return (s - _TGT) / jnp.maximum(k, 1.0)

    d = jax.lax.fori_loop(0, _N_NEWTON_G, newton_g_body, dlo)
    t0 = (m - _TGT) + d

    def newton_f_body(_, t):
        mask = z > t
        k = jnp.sum(mask.astype(z.dtype), axis=-1, keepdims=True)
        s = jnp.sum(jnp.where(mask, z, 0.0), axis=-1, keepdims=True)
        return (s - _TGT) / jnp.maximum(k, 1.0)

    tau = jax.lax.fori_loop(0, _N_NEWTON_F, newton_f_body, t0)

    o_ref[:] = jnp.maximum((z - tau) * (1.0 / _TGT), 0.0)


@jax.jit
def kernel(input):
    b, q, n = input.shape
    rows = b * q
    x2 = input.reshape(rows, n)
    block_rows = 16
    out = pl.pallas_call(
        _sparsegen_rows_kernel,
        out_shape=jax.ShapeDtypeStruct((rows, n), input.dtype),
        grid=(rows // block_rows,),
        in_specs=[pl.BlockSpec((block_rows, n), lambda i: (i, 0))],
        out_specs=pl.BlockSpec((block_rows, n), lambda i: (i, 0)),
    )(x2)
    return out.reshape(b, q, n)


# speedup vs baseline: 114.9566x; 1.1634x over previous
import jax
import jax.numpy as jnp
from jax.experimental import pallas as pl

_LAM = 0.5
_TGT = 1.0 - _LAM
_GRP = 16
_N_BISECT = 12
_N_NEWTON_G = 4
_N_NEWTON_F = 2


def _sparsegen_rows_kernel(x_ref, o_ref):
    z = x_ref[:]
    r, n = z.shape
    g = n // _GRP
    zg = z[:, :g]
    for j in range(1, _GRP):
        zg = jnp.maximum(zg, z[:, j * g:(j + 1) * g])
    m = jnp.max(zg, axis=-1, keepdims=True)
    wg = jnp.maximum(zg - (m - _TGT), 0.0)

    def bisect_body(_, carry):
        dlo, dhi = carry
        mid = 0.5 * (dlo + dhi)
        f = jnp.sum(jnp.maximum(wg - mid, 0.0), axis=-1, keepdims=True)
        gt = f > _TGT
        return jnp.where(gt, mid, dlo), jnp.where(gt, dhi, mid)

    dlo0 = jnp.zeros_like(m)
    dhi0 = jnp.full_like(m, _TGT)
    dlo, _ = jax.lax.fori_loop(0, _N_BISECT, bisect_body, (dlo0, dhi0))

    def newton_g_body(_, d):
        mask = wg > d
        k = jnp.sum(mask.astype(z.dtype), axis=-1, keepdims=True)
        s = jnp.sum(jnp.where(mask, wg, 0.0), axis=-1, keepdims=True)
        return (s - _TGT) / jnp.maximum(k, 1.0)

    d = jax.lax.fori_loop(0, _N_NEWTON_G, newton_g_body, dlo)
    t0 = (m - _TGT) + d

    def newton_f_body(_, t):
        mask = z > t
        k = jnp.sum(mask.astype(z.dtype), axis=-1, keepdims=True)
        s = jnp.sum(jnp.where(mask, z, 0.0), axis=-1, keepdims=True)
        return (s - _TGT) / jnp.maximum(k, 1.0)

    tau = jax.lax.fori_loop(0, _N_NEWTON_F, newton_f_body, t0)

    o_ref[:] = jnp.maximum((z - tau) * (1.0 / _TGT), 0.0)


@jax.jit
def kernel(input):
    b, q, n = input.shape
    rows = b * q
    x2 = input.reshape(rows, n)
    block_rows = 16
    out = pl.pallas_call(
        _sparsegen_rows_kernel,
        out_shape=jax.ShapeDtypeStruct((rows, n), input.dtype),
        grid=(rows // block_rows,),
        in_specs=[pl.BlockSpec((block_rows, n), lambda i: (i, 0))],
        out_specs=pl.BlockSpec((block_rows, n), lambda i: (i, 0)),
    )(x2)
    return out.reshape(b, q, n)
